# baseline (speedup 1.0000x reference)
import jax
import jax.numpy as jnp
from jax import lax
from jax.experimental import pallas as pl
from jax.experimental.pallas import tpu as pltpu


def _round_up(x, m):
    return (x + m - 1) // m * m


def _logits_kernel(x_ref, emb_ref, bias_ref, out_ref):
    tb = out_ref.shape[0]
    b = pl.program_id(1)
    xb = x_ref[pl.ds(b * tb, tb), :].astype(jnp.bfloat16)
    eb = emb_ref[...].astype(jnp.bfloat16)
    acc = lax.dot_general(
        xb, eb,
        dimension_numbers=(((1,), (1,)), ((), ())),
        preferred_element_type=jnp.float32)
    out_ref[...] = acc + bias_ref[...]


def kernel(x, emb_weight, bias):
    B, D = x.shape
    V = emb_weight.shape[0]

    tv = next((t for t in (1280, 640, 512, 768, 384, 256, 128) if V % t == 0), 512)
    V_pad = _round_up(V, tv)
    nv = V_pad // tv

    B_p = _round_up(B, 8)
    x_p = x if B_p == B else jnp.pad(x, ((0, B_p - B), (0, 0)))
    bias_p = bias.astype(jnp.float32)
    if V_pad != V:
        bias_p = jnp.pad(bias_p, ((0, 0), (0, V_pad - V)))

    nb = 2
    tb = B_p // nb
    out = pl.pallas_call(
        _logits_kernel,
        out_shape=jax.ShapeDtypeStruct((B_p, V_pad), jnp.float32),
        grid=(nv, nb),
        in_specs=[
            pl.BlockSpec((B_p, D), lambda v, b: (0, 0)),
            pl.BlockSpec((tv, D), lambda v, b: (v, 0)),
            pl.BlockSpec((1, tv), lambda v, b: (0, v)),
        ],
        out_specs=pl.BlockSpec((tb, tv), lambda v, b: (b, v)),
        compiler_params=pltpu.CompilerParams(
            dimension_semantics=("parallel", "parallel"),
            vmem_limit_bytes=64 * 1024 * 1024,
        ),
    )(x_p, emb_weight, bias_p)
    if B_p != B or V_pad != V:
        out = out[:B, :V]
    return out


# speedup vs baseline: 1.3278x; 1.3278x over previous
import jax
import jax.numpy as jnp
from jax import lax
from jax.experimental import pallas as pl
from jax.experimental.pallas import tpu as pltpu


def _round_up(x, m):
    return (x + m - 1) // m * m


def _logits_kernel(x_ref, emb_ref, bias_ref, out_ref, xb_ref):
    @pl.when(pl.program_id(0) == 0)
    def _():
        xb_ref[...] = x_ref[...].astype(jnp.bfloat16)

    eb = emb_ref[...].astype(jnp.bfloat16)
    acc = lax.dot_general(
        xb_ref[...], eb,
        dimension_numbers=(((1,), (1,)), ((), ())),
        preferred_element_type=jnp.float32)
    out_ref[...] = acc + bias_ref[...]


def kernel(x, emb_weight, bias):
    B, D = x.shape
    V = emb_weight.shape[0]

    tv = next((t for t in (1280, 640, 512, 768, 384, 256, 128) if V % t == 0), 512)
    V_pad = _round_up(V, tv)
    nv = V_pad // tv

    B_p = _round_up(B, 8)
    x_p = x if B_p == B else jnp.pad(x, ((0, B_p - B), (0, 0)))
    bias_p = bias.astype(jnp.float32)
    if V_pad != V:
        bias_p = jnp.pad(bias_p, ((0, 0), (0, V_pad - V)))

    out = pl.pallas_call(
        _logits_kernel,
        out_shape=jax.ShapeDtypeStruct((B_p, V_pad), jnp.float32),
        grid=(nv,),
        in_specs=[
            pl.BlockSpec((B_p, D), lambda v: (0, 0)),
            pl.BlockSpec((tv, D), lambda v: (v, 0)),
            pl.BlockSpec((1, tv), lambda v: (0, v)),
        ],
        out_specs=pl.BlockSpec((B_p, tv), lambda v: (0, v)),
        scratch_shapes=[pltpu.VMEM((B_p, D), jnp.bfloat16)],
        compiler_params=pltpu.CompilerParams(
            dimension_semantics=("arbitrary",),
            vmem_limit_bytes=64 * 1024 * 1024,
        ),
    )(x_p, emb_weight, bias_p)
    if B_p != B or V_pad != V:
        out = out[:B, :V]
    return out


# speedup vs baseline: 1.3537x; 1.0195x over previous
import jax
import jax.numpy as jnp
from jax import lax
from jax.experimental import pallas as pl
from jax.experimental.pallas import tpu as pltpu


def _round_up(x, m):
    return (x + m - 1) // m * m


def _logits_kernel(x_ref, emb_ref, bias_ref, out_ref, xb_ref):
    @pl.when(pl.program_id(0) == 0)
    def _():
        xb_ref[...] = x_ref[...].astype(jnp.bfloat16)

    eb = emb_ref[...].astype(jnp.bfloat16)
    B_p = x_ref.shape[0]
    mc = 1024
    for i in range(0, B_p, mc):
        acc = lax.dot_general(
            xb_ref[i:i + mc, :], eb,
            dimension_numbers=(((1,), (1,)), ((), ())),
            preferred_element_type=jnp.float32)
        out_ref[i:i + mc, :] = acc + bias_ref[...]


def kernel(x, emb_weight, bias):
    B, D = x.shape
    V = emb_weight.shape[0]

    tv = next((t for t in (1280, 640, 512, 768, 384, 256, 128) if V % t == 0), 512)
    V_pad = _round_up(V, tv)
    nv = V_pad // tv

    B_p = _round_up(B, 8)
    x_p = x if B_p == B else jnp.pad(x, ((0, B_p - B), (0, 0)))
    bias_p = bias.astype(jnp.float32)
    if V_pad != V:
        bias_p = jnp.pad(bias_p, ((0, 0), (0, V_pad - V)))

    out = pl.pallas_call(
        _logits_kernel,
        out_shape=jax.ShapeDtypeStruct((B_p, V_pad), jnp.float32),
        grid=(nv,),
        in_specs=[
            pl.BlockSpec((B_p, D), lambda v: (0, 0)),
            pl.BlockSpec((tv, D), lambda v: (v, 0)),
            pl.BlockSpec((1, tv), lambda v: (0, v)),
        ],
        out_specs=pl.BlockSpec((B_p, tv), lambda v: (0, v)),
        scratch_shapes=[pltpu.VMEM((B_p, D), jnp.bfloat16)],
        compiler_params=pltpu.CompilerParams(
            dimension_semantics=("arbitrary",),
            vmem_limit_bytes=64 * 1024 * 1024,
        ),
    )(x_p, emb_weight, bias_p)
    if B_p != B or V_pad != V:
        out = out[:B, :V]
    return out
